# Initial kernel scaffold; baseline (speedup 1.0000x reference)
import functools

import jax
import jax.numpy as jnp
from jax import lax
from jax.experimental import pallas as pl
from jax.experimental.pallas import tpu as pltpu
from jax.experimental.pallas import tpu_sc as plsc

N = 10000
E = 320000
D = 128
NC = 2
NS = 16
NW = NC * NS
NP = 10240
RPT = NP // NW
RSL = NP // NS
EPW = E // NW
CH = 80
NCH = EPW // CH

f32 = jnp.float32

_mesh = plsc.VectorSubcoreMesh(
    core_axis_name="c", subcore_axis_name="s", num_cores=NC, num_subcores=NS)


@functools.partial(
    pl.kernel,
    out_type=(jax.ShapeDtypeStruct((NP, D), f32),
              jax.ShapeDtypeStruct((NC, NP), f32)),
    mesh=_mesh,
    scratch_types=[
        pltpu.VMEM((CH,), jnp.int32),
        pltpu.VMEM((CH, D), f32),
        pltpu.VMEM((NP,), f32),
        pltpu.VMEM((RSL,), f32),
        pltpu.VMEM((RSL,), f32),
        pltpu.VMEM_SHARED((NS, NP), f32),
        pltpu.SemaphoreType.DMA,
    ],
)
def _sc_prep(x_hbm, emb_hbm, dst_hbm, h0_out, cnt_out,
             idx_v, rows_v, cnt_v, tmp_v, acc_v, shared_cnt, sem):
    c = lax.axis_index("c")
    s = lax.axis_index("s")
    wid = s * NC + c

    for k in range(RPT // CH):
        base = wid * RPT + k * CH
        pltpu.sync_copy(x_hbm.at[pl.ds(base, CH)], idx_v)
        pltpu.async_copy(emb_hbm.at[idx_v], rows_v, sem).wait()
        pltpu.sync_copy(rows_v, h0_out.at[pl.ds(base, CH)])

    zeros16 = jnp.zeros((16,), f32)

    def zero_body(i, carry):
        cnt_v[pl.ds(i * 16, 16)] = zeros16
        return carry
    lax.fori_loop(0, NP // 16, zero_body, 0)

    ones16 = jnp.ones((16,), f32)

    def cnt_body(i, carry):
        base = wid * EPW + i * CH
        pltpu.sync_copy(dst_hbm.at[pl.ds(base, CH)], idx_v)
        for j in range(CH // 16):
            d16 = idx_v[pl.ds(j * 16, 16)]
            plsc.addupdate_scatter(cnt_v, [d16], ones16)
        return carry
    lax.fori_loop(0, NCH, cnt_body, 0)

    pltpu.sync_copy(cnt_v, shared_cnt.at[s])
    plsc.subcore_barrier()

    sl = pl.ds(s * RSL, RSL)
    pltpu.sync_copy(shared_cnt.at[0, sl], acc_v)

    def red_body(p, carry):
        pltpu.sync_copy(shared_cnt.at[p, sl], tmp_v)
        for k in range(RSL // 16):
            w = pl.ds(k * 16, 16)
            acc_v[w] = acc_v[w] + tmp_v[w]
        return carry
    lax.fori_loop(1, NS, red_body, 0)
    pltpu.sync_copy(acc_v, cnt_out.at[c, sl])


@functools.partial(
    pl.kernel,
    out_type=jax.ShapeDtypeStruct((NC, NP, D), f32),
    mesh=_mesh,
    scratch_types=[
        pltpu.VMEM((CH,), jnp.int32),
        pltpu.VMEM((CH,), jnp.int32),
        pltpu.VMEM((CH, D), f32),
        pltpu.VMEM_SHARED((NP, D), f32),
        pltpu.SemaphoreType.DMA,
    ],
)
def _sc_edge(hl_hbm, src_hbm, dst_hbm, p_out,
             sidx_v, didx_v, rows_v, accum, sem):
    c = lax.axis_index("c")
    s = lax.axis_index("s")
    wid = s * NC + c

    zeros16 = jnp.zeros((16,), f32)

    def zb(r, carry):
        for j in range(D // 16):
            rows_v[r, pl.ds(j * 16, 16)] = zeros16
        return carry
    lax.fori_loop(0, CH, zb, 0)
    for q in range(RSL // CH):
        pltpu.sync_copy(rows_v, accum.at[pl.ds(s * RSL + q * CH, CH)])
    plsc.subcore_barrier()

    def body(i, carry):
        base = wid * EPW + i * CH
        pltpu.sync_copy(src_hbm.at[pl.ds(base, CH)], sidx_v)
        pltpu.sync_copy(dst_hbm.at[pl.ds(base, CH)], didx_v)
        pltpu.async_copy(hl_hbm.at[sidx_v], rows_v, sem).wait()
        pltpu.sync_copy(rows_v, accum.at[didx_v], add=True)
        return carry
    lax.fori_loop(0, NCH, body, 0)
    plsc.subcore_barrier()

    sl = pl.ds(s * RSL, RSL)
    pltpu.sync_copy(accum.at[sl], p_out.at[c, sl])


R = 1280
G = NP // R


def _dotT(h, w_ref):
    return lax.dot_general(h, w_ref[...], (((1,), (1,)), ((), ())),
                           preferred_element_type=f32)


def _combine(p_ref, cnt_ref, hrr_ref, a_ref):
    psum = p_ref[0] + p_ref[1]
    csum = cnt_ref[0] + cnt_ref[1]
    scale = 1.0 / jnp.maximum(csum, 1.0)
    pre = psum * scale[:, None] + hrr_ref[...]
    av = a_ref[0, 0]
    return jnp.where(pre >= 0, pre, av * pre)


def _tc_first_body(h_ref, wl_ref, wr_ref, bl_ref, br_ref, hl_out, hrr_out):
    h = h_ref[...]
    hl_out[...] = _dotT(h, wl_ref)
    hrr_out[...] = _dotT(h, wr_ref) + bl_ref[...] + br_ref[...]


def _tc_mid_body(p_ref, cnt_ref, hrr_ref, a_ref, wl_ref, wr_ref,
                 bl_ref, br_ref, hl_out, hrr_out):
    h = _combine(p_ref, cnt_ref, hrr_ref, a_ref)
    hl_out[...] = _dotT(h, wl_ref)
    hrr_out[...] = _dotT(h, wr_ref) + bl_ref[...] + br_ref[...]


def _tc_final_body(p_ref, cnt_ref, hrr_ref, a_ref, wout_ref, bout_ref,
                   out_ref):
    h = _combine(p_ref, cnt_ref, hrr_ref, a_ref)
    out_ref[...] = _dotT(h, wout_ref) + bout_ref[...]


_row_spec = pl.BlockSpec((R, D), lambda i: (i, 0))
_w_spec = pl.BlockSpec((D, D), lambda i: (0, 0))
_b_spec = pl.BlockSpec((1, D), lambda i: (0, 0))
_p_spec = pl.BlockSpec((NC, R, D), lambda i: (0, i, 0))
_cnt_spec = pl.BlockSpec((NC, R), lambda i: (0, i))
_a_spec = pl.BlockSpec((1, 1), lambda i: (0, 0))
_hD_shape = jax.ShapeDtypeStruct((NP, D), f32)


def _tc_first(h0, wl, wr, bl, br):
    return pl.pallas_call(
        _tc_first_body,
        grid=(G,),
        in_specs=[_row_spec, _w_spec, _w_spec, _b_spec, _b_spec],
        out_specs=[_row_spec, _row_spec],
        out_shape=[_hD_shape, _hD_shape],
    )(h0, wl, wr, bl.reshape(1, D), br.reshape(1, D))


def _tc_mid(p, cnt, hrr, a, wl, wr, bl, br):
    return pl.pallas_call(
        _tc_mid_body,
        grid=(G,),
        in_specs=[_p_spec, _cnt_spec, _row_spec, _a_spec,
                  _w_spec, _w_spec, _b_spec, _b_spec],
        out_specs=[_row_spec, _row_spec],
        out_shape=[_hD_shape, _hD_shape],
    )(p, cnt, hrr, a.reshape(1, 1), wl, wr,
      bl.reshape(1, D), br.reshape(1, D))


def _tc_final(p, cnt, hrr, a, wout, bout):
    return pl.pallas_call(
        _tc_final_body,
        grid=(G,),
        in_specs=[_p_spec, _cnt_spec, _row_spec, _a_spec, _w_spec, _b_spec],
        out_specs=_row_spec,
        out_shape=_hD_shape,
    )(p, cnt, hrr, a.reshape(1, 1), wout, bout.reshape(1, D))


def kernel(x, edge_index, edge_weight, emb,
           Wl1, bl1, Wr1, br1, a1,
           Wl2, bl2, Wr2, br2, a2,
           Wl3, bl3, Wr3, br3, a3,
           Wout, bout):
    x_pad = jnp.pad(x.astype(jnp.int32), (0, NP - N))
    src = edge_index[0].astype(jnp.int32)
    dst = edge_index[1].astype(jnp.int32)

    h0, cnt = _sc_prep(x_pad, emb.astype(f32), dst)
    hl, hrr = _tc_first(h0, Wl1, Wr1, bl1, br1)
    p = _sc_edge(hl, src, dst)
    hl, hrr = _tc_mid(p, cnt, hrr, a1, Wl2, Wr2, bl2, br2)
    p = _sc_edge(hl, src, dst)
    hl, hrr = _tc_mid(p, cnt, hrr, a2, Wl3, Wr3, bl3, br3)
    p = _sc_edge(hl, src, dst)
    out = _tc_final(p, cnt, hrr, a3, Wout, bout)
    return out[:N]


# speedup vs baseline: 4.4480x; 4.4480x over previous
import functools

import jax
import jax.numpy as jnp
from jax import lax
from jax.experimental import pallas as pl
from jax.experimental.pallas import tpu as pltpu
from jax.experimental.pallas import tpu_sc as plsc

N = 10000
E = 320000
D = 128
NC = 2
NS = 16
NW = NC * NS
NP = 10240
RPT = NP // NW
RSL = NP // NS
EPW = E // NW
CH = 80
NCH = EPW // CH

f32 = jnp.float32

_mesh = plsc.VectorSubcoreMesh(
    core_axis_name="c", subcore_axis_name="s", num_cores=NC, num_subcores=NS)


@functools.partial(
    pl.kernel,
    out_type=(jax.ShapeDtypeStruct((NP, D), f32),
              jax.ShapeDtypeStruct((NC, NP), f32)),
    mesh=_mesh,
    scratch_types=[
        pltpu.VMEM((CH,), jnp.int32),
        pltpu.VMEM((CH, D), f32),
        pltpu.VMEM((NP,), f32),
        pltpu.VMEM((RSL,), f32),
        pltpu.VMEM((RSL,), f32),
        pltpu.VMEM_SHARED((NS, NP), f32),
        pltpu.SemaphoreType.DMA,
    ],
    compiler_params=pltpu.CompilerParams(needs_layout_passes=False),
)
def _sc_prep(x_hbm, emb_hbm, dst_hbm, h0_out, cnt_out,
             idx_v, rows_v, cnt_v, tmp_v, acc_v, shared_cnt, sem):
    c = lax.axis_index("c")
    s = lax.axis_index("s")
    wid = s * NC + c

    for k in range(RPT // CH):
        base = wid * RPT + k * CH
        pltpu.sync_copy(x_hbm.at[pl.ds(base, CH)], idx_v)
        pltpu.async_copy(emb_hbm.at[idx_v], rows_v, sem).wait()
        pltpu.sync_copy(rows_v, h0_out.at[pl.ds(base, CH)])

    zeros16 = jnp.zeros((16,), f32)

    def zero_body(i, carry):
        cnt_v[pl.ds(i * 16, 16)] = zeros16
        return carry
    lax.fori_loop(0, NP // 16, zero_body, 0)

    ones16 = jnp.ones((16,), f32)

    def cnt_body(i, carry):
        base = wid * EPW + i * CH
        pltpu.sync_copy(dst_hbm.at[pl.ds(base, CH)], idx_v)
        for j in range(CH // 16):
            d16 = idx_v[pl.ds(j * 16, 16)]
            plsc.addupdate_scatter(cnt_v, [d16], ones16)
        return carry
    lax.fori_loop(0, NCH, cnt_body, 0)

    pltpu.sync_copy(cnt_v, shared_cnt.at[s])
    plsc.subcore_barrier()

    sl = pl.ds(s * RSL, RSL)
    pltpu.sync_copy(shared_cnt.at[0, sl], acc_v)

    def red_body(p, carry):
        pltpu.sync_copy(shared_cnt.at[p, sl], tmp_v)
        for k in range(RSL // 16):
            w = pl.ds(k * 16, 16)
            acc_v[w] = acc_v[w] + tmp_v[w]
        return carry
    lax.fori_loop(1, NS, red_body, 0)
    pltpu.sync_copy(acc_v, cnt_out.at[c, sl])


@functools.partial(
    pl.kernel,
    out_type=jax.ShapeDtypeStruct((NC, NP, D), f32),
    mesh=_mesh,
    scratch_types=[
        pltpu.VMEM((CH,), jnp.int32),
        pltpu.VMEM((CH,), jnp.int32),
        pltpu.VMEM((CH, D), f32),
        pltpu.VMEM_SHARED((NP, D), f32),
        pltpu.SemaphoreType.DMA,
    ],
    compiler_params=pltpu.CompilerParams(needs_layout_passes=False),
)
def _sc_edge(hl_hbm, src_hbm, dst_hbm, p_out,
             sidx_v, didx_v, rows_v, accum, sem):
    c = lax.axis_index("c")
    s = lax.axis_index("s")
    wid = s * NC + c

    zeros16 = jnp.zeros((16,), f32)

    def zb(r, carry):
        for j in range(D // 16):
            rows_v[r, pl.ds(j * 16, 16)] = zeros16
        return carry
    lax.fori_loop(0, CH, zb, 0)
    for q in range(RSL // CH):
        pltpu.sync_copy(rows_v, accum.at[pl.ds(s * RSL + q * CH, CH)])
    plsc.subcore_barrier()

    def body(i, carry):
        base = wid * EPW + i * CH
        pltpu.sync_copy(src_hbm.at[pl.ds(base, CH)], sidx_v)
        pltpu.sync_copy(dst_hbm.at[pl.ds(base, CH)], didx_v)
        pltpu.async_copy(hl_hbm.at[sidx_v], rows_v, sem).wait()
        pltpu.sync_copy(rows_v, accum.at[didx_v], add=True)
        return carry
    lax.fori_loop(0, NCH, body, 0)
    plsc.subcore_barrier()

    sl = pl.ds(s * RSL, RSL)
    pltpu.sync_copy(accum.at[sl], p_out.at[c, sl])


R = 1280
G = NP // R


def _dotT(h, w_ref):
    return lax.dot_general(h, w_ref[...], (((1,), (1,)), ((), ())),
                           preferred_element_type=f32)


def _combine(p_ref, cnt_ref, hrr_ref, a_ref):
    psum = p_ref[0] + p_ref[1]
    csum = cnt_ref[0] + cnt_ref[1]
    scale = 1.0 / jnp.maximum(csum, 1.0)
    pre = psum * scale[:, None] + hrr_ref[...]
    av = a_ref[0, 0]
    return jnp.where(pre >= 0, pre, av * pre)


def _tc_first_body(h_ref, wl_ref, wr_ref, bl_ref, br_ref, hl_out, hrr_out):
    h = h_ref[...]
    hl_out[...] = _dotT(h, wl_ref)
    hrr_out[...] = _dotT(h, wr_ref) + bl_ref[...] + br_ref[...]


def _tc_mid_body(p_ref, cnt_ref, hrr_ref, a_ref, wl_ref, wr_ref,
                 bl_ref, br_ref, hl_out, hrr_out):
    h = _combine(p_ref, cnt_ref, hrr_ref, a_ref)
    hl_out[...] = _dotT(h, wl_ref)
    hrr_out[...] = _dotT(h, wr_ref) + bl_ref[...] + br_ref[...]


def _tc_final_body(p_ref, cnt_ref, hrr_ref, a_ref, wout_ref, bout_ref,
                   out_ref):
    h = _combine(p_ref, cnt_ref, hrr_ref, a_ref)
    out_ref[...] = _dotT(h, wout_ref) + bout_ref[...]


_row_spec = pl.BlockSpec((R, D), lambda i: (i, 0))
_w_spec = pl.BlockSpec((D, D), lambda i: (0, 0))
_b_spec = pl.BlockSpec((1, D), lambda i: (0, 0))
_p_spec = pl.BlockSpec((NC, R, D), lambda i: (0, i, 0))
_cnt_spec = pl.BlockSpec((NC, R), lambda i: (0, i))
_a_spec = pl.BlockSpec((1, 1), lambda i: (0, 0))
_hD_shape = jax.ShapeDtypeStruct((NP, D), f32)


def _tc_first(h0, wl, wr, bl, br):
    return pl.pallas_call(
        _tc_first_body,
        grid=(G,),
        in_specs=[_row_spec, _w_spec, _w_spec, _b_spec, _b_spec],
        out_specs=[_row_spec, _row_spec],
        out_shape=[_hD_shape, _hD_shape],
    )(h0, wl, wr, bl.reshape(1, D), br.reshape(1, D))


def _tc_mid(p, cnt, hrr, a, wl, wr, bl, br):
    return pl.pallas_call(
        _tc_mid_body,
        grid=(G,),
        in_specs=[_p_spec, _cnt_spec, _row_spec, _a_spec,
                  _w_spec, _w_spec, _b_spec, _b_spec],
        out_specs=[_row_spec, _row_spec],
        out_shape=[_hD_shape, _hD_shape],
    )(p, cnt, hrr, a.reshape(1, 1), wl, wr,
      bl.reshape(1, D), br.reshape(1, D))


def _tc_final(p, cnt, hrr, a, wout, bout):
    return pl.pallas_call(
        _tc_final_body,
        grid=(G,),
        in_specs=[_p_spec, _cnt_spec, _row_spec, _a_spec, _w_spec, _b_spec],
        out_specs=_row_spec,
        out_shape=_hD_shape,
    )(p, cnt, hrr, a.reshape(1, 1), wout, bout.reshape(1, D))


def kernel(x, edge_index, edge_weight, emb,
           Wl1, bl1, Wr1, br1, a1,
           Wl2, bl2, Wr2, br2, a2,
           Wl3, bl3, Wr3, br3, a3,
           Wout, bout):
    x_pad = jnp.pad(x.astype(jnp.int32), (0, NP - N))
    src = edge_index[0].astype(jnp.int32)
    dst = edge_index[1].astype(jnp.int32)

    h0, cnt = _sc_prep(x_pad, emb.astype(f32), dst)
    hl, hrr = _tc_first(h0, Wl1, Wr1, bl1, br1)
    p = _sc_edge(hl, src, dst)
    hl, hrr = _tc_mid(p, cnt, hrr, a1, Wl2, Wr2, bl2, br2)
    p = _sc_edge(hl, src, dst)
    hl, hrr = _tc_mid(p, cnt, hrr, a2, Wl3, Wr3, bl3, br3)
    p = _sc_edge(hl, src, dst)
    out = _tc_final(p, cnt, hrr, a3, Wout, bout)
    return out[:N]


# speedup vs baseline: 10.5386x; 2.3693x over previous
import functools

import jax
import jax.numpy as jnp
from jax import lax
from jax.experimental import pallas as pl
from jax.experimental.pallas import tpu as pltpu
from jax.experimental.pallas import tpu_sc as plsc

N = 10000
E = 320000
D = 128
NC = 2
NS = 16
NW = NC * NS
NP = 10240
RPT = NP // NW
RSL = NP // NS
EPW = E // NW
CH = 80
NCH = EPW // CH

f32 = jnp.float32

_mesh = plsc.VectorSubcoreMesh(
    core_axis_name="c", subcore_axis_name="s", num_cores=NC, num_subcores=NS)


@functools.partial(
    pl.kernel,
    out_type=(jax.ShapeDtypeStruct((NP, D), f32),
              jax.ShapeDtypeStruct((NC, NP), f32)),
    mesh=_mesh,
    scratch_types=[
        pltpu.VMEM((RPT // CH, CH), jnp.int32),
        pltpu.VMEM((CH, D), f32),
        pltpu.VMEM((CH, D), f32),
        pltpu.VMEM((NCH, CH), jnp.int32),
        pltpu.VMEM((NP,), f32),
        pltpu.VMEM((RSL,), f32),
        pltpu.VMEM((RSL,), f32),
        pltpu.VMEM_SHARED((NS, NP), f32),
        pltpu.SemaphoreType.DMA,
        pltpu.SemaphoreType.DMA,
    ],
    compiler_params=pltpu.CompilerParams(needs_layout_passes=False),
)
def _sc_prep(x_hbm, emb_hbm, dst2_hbm, h0_out, cnt_out,
             xidx2, rows_a, rows_b, didx2, cnt_v, tmp_v, acc_v,
             shared_cnt, sem_a, sem_b):
    c = lax.axis_index("c")
    s = lax.axis_index("s")
    wid = s * NC + c
    kch = RPT // CH

    pltpu.sync_copy(x_hbm.at[wid], xidx2)
    pltpu.sync_copy(dst2_hbm.at[wid], didx2)

    bufs = (rows_a, rows_b)
    sems = (sem_a, sem_b)
    for k in range(kch):
        pltpu.async_copy(emb_hbm.at[xidx2.at[k]], bufs[k % 2], sems[k % 2])
        if k >= 1:
            pltpu.make_async_copy(emb_hbm.at[xidx2.at[k - 1]],
                                  bufs[(k - 1) % 2], sems[(k - 1) % 2]).wait()
            pltpu.sync_copy(bufs[(k - 1) % 2],
                            h0_out.at[pl.ds(wid * RPT + (k - 1) * CH, CH)])
    pltpu.make_async_copy(emb_hbm.at[xidx2.at[kch - 1]],
                          bufs[(kch - 1) % 2], sems[(kch - 1) % 2]).wait()
    pltpu.sync_copy(bufs[(kch - 1) % 2],
                    h0_out.at[pl.ds(wid * RPT + (kch - 1) * CH, CH)])

    zeros16 = jnp.zeros((16,), f32)

    def zero_body(i, carry):
        cnt_v[pl.ds(i * 16, 16)] = zeros16
        return carry
    lax.fori_loop(0, NP // 16, zero_body, 0)

    ones16 = jnp.ones((16,), f32)

    def cnt_body(i, carry):
        for j in range(CH // 16):
            d16 = didx2[i, pl.ds(j * 16, 16)]
            plsc.addupdate_scatter(cnt_v, [d16], ones16)
        return carry
    lax.fori_loop(0, NCH, cnt_body, 0)

    pltpu.sync_copy(cnt_v, shared_cnt.at[s])
    plsc.subcore_barrier()

    sl = pl.ds(s * RSL, RSL)
    pltpu.sync_copy(shared_cnt.at[0, sl], acc_v)

    def red_body(p, carry):
        pltpu.sync_copy(shared_cnt.at[p, sl], tmp_v)
        for k in range(RSL // 16):
            w = pl.ds(k * 16, 16)
            acc_v[w] = acc_v[w] + tmp_v[w]
        return carry
    lax.fori_loop(1, NS, red_body, 0)
    pltpu.sync_copy(acc_v, cnt_out.at[c, sl])


NRB = 2
NIB = 4


@functools.partial(
    pl.kernel,
    out_type=jax.ShapeDtypeStruct((NC, NP, D), f32),
    mesh=_mesh,
    scratch_types=[
        pltpu.VMEM((NIB, CH), jnp.int32),
        pltpu.VMEM((NIB, CH), jnp.int32),
        pltpu.VMEM((NRB, CH, D), f32),
        pltpu.VMEM_SHARED((NP, D), f32),
        pltpu.SemaphoreType.DMA,
        pltpu.SemaphoreType.DMA,
        pltpu.SemaphoreType.DMA,
        pltpu.SemaphoreType.DMA,
        pltpu.SemaphoreType.DMA,
        pltpu.SemaphoreType.DMA,
    ],
    compiler_params=pltpu.CompilerParams(needs_layout_passes=False),
)
def _sc_edge(hl_hbm, src_hbm, dst_hbm, p_out,
             sidx, didx, rows, accum, g0, g1, i0, i1, i2, i3):
    c = lax.axis_index("c")
    s = lax.axis_index("s")
    wid = s * NC + c
    gsems = (g0, g1)
    isems = (i0, i1, i2, i3)

    zeros16 = jnp.zeros((16,), f32)

    def zb(r, carry):
        for j in range(D // 16):
            rows[0, r, pl.ds(j * 16, 16)] = zeros16
        return carry
    lax.fori_loop(0, CH, zb, 0)
    for q in range(RSL // CH):
        pltpu.sync_copy(rows.at[0], accum.at[pl.ds(s * RSL + q * CH, CH)])
    plsc.subcore_barrier()

    def idx_start(i, j):
        base = wid * EPW + i * CH
        pltpu.async_copy(src_hbm.at[pl.ds(base, CH)], sidx.at[j], isems[j])
        pltpu.async_copy(dst_hbm.at[pl.ds(base, CH)], didx.at[j], isems[j])

    def idx_wait(i, j):
        base = wid * EPW + i * CH
        pltpu.make_async_copy(src_hbm.at[pl.ds(base, CH)], sidx.at[j],
                              isems[j]).wait()
        pltpu.make_async_copy(dst_hbm.at[pl.ds(base, CH)], didx.at[j],
                              isems[j]).wait()

    def gather_start(i, jr, ji):
        pltpu.async_copy(hl_hbm.at[sidx.at[ji]], rows.at[jr], gsems[jr])

    def gather_wait(i, jr, ji):
        pltpu.make_async_copy(hl_hbm.at[sidx.at[ji]], rows.at[jr],
                              gsems[jr]).wait()

    for j in range(NIB):
        idx_start(j, j)
    for j in range(NRB):
        idx_wait(j, j)
        gather_start(j, j, j)

    def body(k, carry):
        for j in range(NIB):
            cur = NIB * k + j
            gather_wait(cur, j % NRB, j)
            pltpu.sync_copy(rows.at[j % NRB], accum.at[didx.at[j]],
                            add=True)
            nxt_i = cur + NIB

            @pl.when(nxt_i < NCH)
            def _():
                idx_start(nxt_i, j)
            nxt_g = cur + NRB

            @pl.when(nxt_g < NCH)
            def _():
                idx_wait(nxt_g, (j + NRB) % NIB)
                gather_start(nxt_g, j % NRB, (j + NRB) % NIB)
        return carry
    lax.fori_loop(0, NCH // NIB, body, 0)
    gather_wait(NCH - 1, (NCH - 1) % NRB, (NCH - 1) % NIB)
    pltpu.sync_copy(rows.at[(NCH - 1) % NRB],
                    accum.at[didx.at[(NCH - 1) % NIB]], add=True)
    plsc.subcore_barrier()

    sl = pl.ds(s * RSL, RSL)
    pltpu.sync_copy(accum.at[sl], p_out.at[c, sl])


R = 1280
G = NP // R


def _dotT(h, w_ref):
    return lax.dot_general(h, w_ref[...], (((1,), (1,)), ((), ())),
                           preferred_element_type=f32)


def _combine(p_ref, cnt_ref, hrr_ref, a_ref):
    psum = p_ref[0] + p_ref[1]
    csum = cnt_ref[0] + cnt_ref[1]
    scale = 1.0 / jnp.maximum(csum, 1.0)
    pre = psum * scale[:, None] + hrr_ref[...]
    av = a_ref[0, 0]
    return jnp.where(pre >= 0, pre, av * pre)


def _tc_first_body(h_ref, wl_ref, wr_ref, bl_ref, br_ref, hl_out, hrr_out):
    h = h_ref[...]
    hl_out[...] = _dotT(h, wl_ref)
    hrr_out[...] = _dotT(h, wr_ref) + bl_ref[...] + br_ref[...]


def _tc_mid_body(p_ref, cnt_ref, hrr_ref, a_ref, wl_ref, wr_ref,
                 bl_ref, br_ref, hl_out, hrr_out):
    h = _combine(p_ref, cnt_ref, hrr_ref, a_ref)
    hl_out[...] = _dotT(h, wl_ref)
    hrr_out[...] = _dotT(h, wr_ref) + bl_ref[...] + br_ref[...]


def _tc_final_body(p_ref, cnt_ref, hrr_ref, a_ref, wout_ref, bout_ref,
                   out_ref):
    h = _combine(p_ref, cnt_ref, hrr_ref, a_ref)
    out_ref[...] = _dotT(h, wout_ref) + bout_ref[...]


_row_spec = pl.BlockSpec((R, D), lambda i: (i, 0))
_w_spec = pl.BlockSpec((D, D), lambda i: (0, 0))
_b_spec = pl.BlockSpec((1, D), lambda i: (0, 0))
_p_spec = pl.BlockSpec((NC, R, D), lambda i: (0, i, 0))
_cnt_spec = pl.BlockSpec((NC, R), lambda i: (0, i))
_a_spec = pl.BlockSpec((1, 1), lambda i: (0, 0))
_hD_shape = jax.ShapeDtypeStruct((NP, D), f32)


def _tc_first(h0, wl, wr, bl, br):
    return pl.pallas_call(
        _tc_first_body,
        grid=(G,),
        in_specs=[_row_spec, _w_spec, _w_spec, _b_spec, _b_spec],
        out_specs=[_row_spec, _row_spec],
        out_shape=[_hD_shape, _hD_shape],
    )(h0, wl, wr, bl.reshape(1, D), br.reshape(1, D))


def _tc_mid(p, cnt, hrr, a, wl, wr, bl, br):
    return pl.pallas_call(
        _tc_mid_body,
        grid=(G,),
        in_specs=[_p_spec, _cnt_spec, _row_spec, _a_spec,
                  _w_spec, _w_spec, _b_spec, _b_spec],
        out_specs=[_row_spec, _row_spec],
        out_shape=[_hD_shape, _hD_shape],
    )(p, cnt, hrr, a.reshape(1, 1), wl, wr,
      bl.reshape(1, D), br.reshape(1, D))


def _tc_final(p, cnt, hrr, a, wout, bout):
    return pl.pallas_call(
        _tc_final_body,
        grid=(G,),
        in_specs=[_p_spec, _cnt_spec, _row_spec, _a_spec, _w_spec, _b_spec],
        out_specs=_row_spec,
        out_shape=_hD_shape,
    )(p, cnt, hrr, a.reshape(1, 1), wout, bout.reshape(1, D))


def kernel(x, edge_index, edge_weight, emb,
           Wl1, bl1, Wr1, br1, a1,
           Wl2, bl2, Wr2, br2, a2,
           Wl3, bl3, Wr3, br3, a3,
           Wout, bout):
    x2 = jnp.pad(x.astype(jnp.int32), (0, NP - N)).reshape(NW, -1, CH)
    src = edge_index[0].astype(jnp.int32)
    dst = edge_index[1].astype(jnp.int32)
    dst2 = dst.reshape(NW, NCH, CH)

    h0, cnt = _sc_prep(x2, emb.astype(f32), dst2)
    hl, hrr = _tc_first(h0, Wl1, Wr1, bl1, br1)
    p = _sc_edge(hl, src, dst)
    hl, hrr = _tc_mid(p, cnt, hrr, a1, Wl2, Wr2, bl2, br2)
    p = _sc_edge(hl, src, dst)
    hl, hrr = _tc_mid(p, cnt, hrr, a2, Wl3, Wr3, bl3, br3)
    p = _sc_edge(hl, src, dst)
    out = _tc_final(p, cnt, hrr, a3, Wout, bout)
    return out[:N]
